# baseline (speedup 1.0000x reference)
import jax
import jax.numpy as jnp
from jax.experimental import pallas as pl


def _body(x_ref, pos_ref, out_ref):
    bb, s, d = x_ref.shape
    e = pos_ref.shape[-1]
    out_ref[:, :, :d] = x_ref[...]
    out_ref[:, :, d:] = jnp.broadcast_to(pos_ref[...][None, :, :], (bb, s, e))


def kernel(x, pos_table):
    B, S, D = x.shape
    E = pos_table.shape[-1]
    BB = 64
    return pl.pallas_call(
        _body,
        grid=(B // BB,),
        in_specs=[
            pl.BlockSpec((BB, S, D), lambda i: (i, 0, 0)),
            pl.BlockSpec((S, E), lambda i: (0, 0)),
        ],
        out_specs=pl.BlockSpec((BB, S, D + E), lambda i: (i, 0, 0)),
        out_shape=jax.ShapeDtypeStruct((B, S, D + E), x.dtype),
    )(x, pos_table)
